# Initial kernel scaffold; baseline (speedup 1.0000x reference)
import jax
import jax.numpy as jnp
from jax.experimental import pallas as pl


def kernel(pos, atom_emb, Wn, bn, Wt1, bt1, Wt2, bt2, q_emb, Wq, bq, We1, be1, We2, be2, Win, bin_, Wm1, bm1, Wm2, bm2, Watt, batt, Wx, bx, Wh1, bh1, Wh2, bh2, atom_type, edge_index, node_batch, is_query, t):
    raise NotImplementedError("write your pallas kernel here")


# speedup vs baseline: 1.9229x; 1.9229x over previous
import functools
import math

import jax
import jax.numpy as jnp
from jax import lax
from jax.experimental import pallas as pl
from jax.experimental.pallas import tpu as pltpu
from jax.experimental.pallas import tpu_sc as plsc

N = 10000
E = 320000
HID = 64
XP = 16
ACC = HID + XP
AVG_DEG = 32.0

NC, NS = 2, 16
NW = NC * NS
EPW = E // NW
CH = 80
NCHUNK = EPW // CH
NPC = N // NS

BE = 2000
BN = 1000

f32 = jnp.float32


def _silu(x):
    return x * jax.nn.sigmoid(x)


@functools.cache
def _sc_kernels():
    mesh = plsc.VectorSubcoreMesh(core_axis_name="c", subcore_axis_name="s")
    cp = pltpu.CompilerParams(use_tc_tiling_on_sc=False)

    @functools.partial(
        pl.kernel,
        compiler_params=cp,
        out_type=(
            jax.ShapeDtypeStruct((E, HID), f32),
            jax.ShapeDtypeStruct((E, HID), f32),
            jax.ShapeDtypeStruct((E, XP), f32),
            jax.ShapeDtypeStruct((E, XP), f32),
        ),
        mesh=mesh,
        scratch_types=[
            pltpu.VMEM((CH,), jnp.int32),
            pltpu.VMEM((CH, HID), f32),
            pltpu.VMEM((CH, XP), f32),
            pltpu.SemaphoreType.DMA,
        ],
    )
    def sc_gather(pha, phb, xpad, src, dst, hs, hd, xs, xd, idx_v, b64, b16,
                  sem):
        wid = lax.axis_index("s") * NC + lax.axis_index("c")
        base = wid * EPW

        @pl.loop(0, NCHUNK)
        def _(i):
            off = base + i * CH
            pltpu.sync_copy(src.at[pl.ds(off, CH)], idx_v)
            pltpu.async_copy(pha.at[idx_v], b64, sem).wait()
            pltpu.sync_copy(b64, hs.at[pl.ds(off, CH)])
            pltpu.async_copy(xpad.at[idx_v], b16, sem).wait()
            pltpu.sync_copy(b16, xs.at[pl.ds(off, CH)])
            pltpu.sync_copy(dst.at[pl.ds(off, CH)], idx_v)
            pltpu.async_copy(phb.at[idx_v], b64, sem).wait()
            pltpu.sync_copy(b64, hd.at[pl.ds(off, CH)])
            pltpu.async_copy(xpad.at[idx_v], b16, sem).wait()
            pltpu.sync_copy(b16, xd.at[pl.ds(off, CH)])

    @functools.partial(
        pl.kernel,
        compiler_params=cp,
        out_type=jax.ShapeDtypeStruct((NC, N, ACC), f32),
        mesh=mesh,
        scratch_types=[
            pltpu.VMEM((CH,), jnp.int32),
            pltpu.VMEM((CH, ACC), f32),
            pltpu.VMEM_SHARED((N, ACC), f32),
            pltpu.SemaphoreType.DMA,
        ],
    )
    def sc_scatter(macc, dstidx, zeros, acc_out, idx_v, mbuf, acc_sh, sem):
        cid = lax.axis_index("c")
        sid = lax.axis_index("s")
        wid = sid * NC + cid
        pltpu.sync_copy(zeros.at[pl.ds(sid * NPC, NPC)],
                        acc_sh.at[pl.ds(sid * NPC, NPC)])
        plsc.subcore_barrier()
        base = wid * EPW

        @pl.loop(0, NCHUNK)
        def _(i):
            off = base + i * CH
            pltpu.sync_copy(dstidx.at[pl.ds(off, CH)], idx_v)
            pltpu.sync_copy(macc.at[pl.ds(off, CH)], mbuf)
            pltpu.sync_copy(mbuf, acc_sh.at[idx_v], add=True)

        plsc.subcore_barrier()
        pltpu.sync_copy(acc_sh.at[pl.ds(sid * NPC, NPC)],
                        acc_out.at[cid].at[pl.ds(sid * NPC, NPC)])

    return sc_gather, sc_scatter


def _full(shape):
    return pl.BlockSpec(shape, lambda i: tuple(0 for _ in shape))


def _blk(d):
    return pl.BlockSpec((BE, d), lambda i: (i, 0))


def _nblk(d):
    return pl.BlockSpec((BN, d), lambda i: (i, 0))


def _bf(x):
    return x.astype(jnp.bfloat16)


def _dot16(a, b):
    return jnp.dot(_bf(a), _bf(b), preferred_element_type=f32)


def _edge_body(first, hs, hd, xs, xd, ear, we1, be1, We2, Wd, wd2, bm1r,
               be2, Wm2, bm2r, Watt, battr, Wx, bxr, macc, eaout):
    diff = xs[...] - xd[...]
    d2 = jnp.sum(diff * diff, axis=1, keepdims=True)
    if first:
        d0 = jnp.sqrt(d2 + 1e-8)
        u = _silu(d0 * we1[...] + be1[...])
        ea = _bf(_dot16(u, We2[...]) + be2[...])
        eaout[...] = ea
    else:
        ea = ear[...]
    d2c = _bf(d2).astype(f32) * _bf(wd2[...]).astype(f32)
    pre1 = (hs[...] + hd[...] + d2c
            + jnp.dot(ea, _bf(Wd[...]), preferred_element_type=f32)
            + bm1r[...])
    m = _silu(_dot16(_silu(pre1), Wm2[...]) + bm2r[...])
    att = jax.nn.sigmoid(_dot16(m, Watt[...]) + battr[...])
    mt = m * att
    coef = jnp.tanh(_dot16(mt, Wx[...]) + bxr[...])
    macc[...] = jnp.concatenate([mt, diff * coef], axis=1)


def _edge_call(first, hs, hd, xs, xd, ea16, we1r, be1r, We2, Wd, wd2r, bm1r,
               be2r, Wm2, bm2r, Watt, battr, Wx, bxr):
    grid = (E // BE,)
    win = [_blk(HID), _blk(HID), _blk(XP), _blk(XP)]
    args = [hs, hd, xs, xd]
    if not first:
        win.append(_blk(HID))
        args.append(ea16)
    win += [_full((1, HID)), _full((1, HID)), _full((HID, HID)),
            _full((HID, HID)), _full((1, HID)), _full((1, HID)),
            _full((1, HID)), _full((HID, HID)), _full((1, HID)),
            _full((HID, 1)), _full((1, 1)), _full((HID, 1)), _full((1, 1))]
    args += [we1r, be1r, We2, Wd, wd2r, bm1r, be2r, Wm2, bm2r, Watt, battr,
             Wx, bxr]
    out_shape = [jax.ShapeDtypeStruct((E, ACC), f32)]
    out_specs = [_blk(ACC)]
    if first:
        out_shape.append(jax.ShapeDtypeStruct((E, HID), jnp.bfloat16))
        out_specs.append(_blk(HID))

    def body(*refs):
        ins = refs[:4]
        k = 4
        ear = None
        if not first:
            ear = refs[4]
            k = 5
        w = refs[k:k + 13]
        outs = refs[k + 13:]
        _edge_body(first, *ins, ear, *w, outs[0],
                   outs[1] if first else None)

    res = pl.pallas_call(
        body, grid=grid, in_specs=win, out_specs=out_specs,
        out_shape=out_shape)(*args)
    return res if first else (res[0], None)


def _node_body(last, h, xpad, acc0, acc1, qm, pospad, Wh1h, Wh1a, bh1r, Wh2,
               bh2r, Wa, Wb, h_o, x_o, pha_o, phb_o, vt_o):
    dx = ((acc0[..., HID:] + acc1[..., HID:]) * (1.0 / AVG_DEG)) * qm[...]
    x_new = xpad[...] + dx
    if last:
        vt_o[...] = x_new - pospad[...]
        return
    agg = acc0[..., :HID] + acc1[..., :HID]
    hh = _silu(_dot16(h[...], Wh1h[...]) + _dot16(agg, Wh1a[...]) + bh1r[...])
    h_new = h[...] + _dot16(hh, Wh2[...]) + bh2r[...]
    h_o[...] = h_new
    x_o[...] = x_new
    pha_o[...] = _dot16(h_new, Wa[...])
    phb_o[...] = _dot16(h_new, Wb[...])


def _node_call(last, h, xpad, acc0, acc1, qm, pospad, Wh1h, Wh1a, bh1r, Wh2,
               bh2r, Wa, Wb):
    grid = (N // BN,)
    if last:
        args = [xpad, acc0, acc1, qm, pospad]
        win = [_nblk(XP), _nblk(ACC), _nblk(ACC), _nblk(1), _nblk(XP)]
        out_shape = jax.ShapeDtypeStruct((N, XP), f32)
        out_specs = _nblk(XP)

        def body(xr, a0, a1, qr, pr, vt):
            _node_body(True, None, xr, a0, a1, qr, pr, None, None, None,
                       None, None, None, None, None, None, None, None, vt)
    else:
        args = [h, xpad, acc0, acc1, qm, Wh1h, Wh1a, bh1r, Wh2, bh2r, Wa, Wb]
        win = [_nblk(HID), _nblk(XP), _nblk(ACC), _nblk(ACC), _nblk(1),
               _full((HID, HID)), _full((HID, HID)), _full((1, HID)),
               _full((HID, HID)), _full((1, HID)), _full((HID, HID)),
               _full((HID, HID))]
        out_shape = [jax.ShapeDtypeStruct((N, HID), f32),
                     jax.ShapeDtypeStruct((N, XP), f32),
                     jax.ShapeDtypeStruct((N, HID), f32),
                     jax.ShapeDtypeStruct((N, HID), f32)]
        out_specs = [_nblk(HID), _nblk(XP), _nblk(HID), _nblk(HID)]

        def body(hr, xr, a0, a1, qr, w1h, w1a, b1, w2, b2, wa, wb,
                 ho, xo, pa, pb):
            _node_body(False, hr, xr, a0, a1, qr, None, w1h, w1a, b1, w2,
                       b2, wa, wb, ho, xo, pa, pb, None)

    return pl.pallas_call(body, grid=grid, in_specs=win, out_specs=out_specs,
                          out_shape=out_shape)(*args)


def _prep_call(atom_t, nb, isq, tcol, atom_emb, Wn, bnr, Wt1, bt1r, Wt2,
               bt2r, q_emb, Wq, bqr, Win, binr, Wa0, Wb0):
    grid = (N // BN,)
    win = [_nblk(1), _nblk(1), _nblk(1), _full((32, 1)),
           _full((100, HID)), _full((HID, HID)), _full((1, HID)),
           _full((32, 32)), _full((1, 32)), _full((32, 32)), _full((1, 32)),
           _full((2, 32)), _full((32, 32)), _full((1, 32)),
           _full((128, HID)), _full((1, HID)),
           _full((HID, HID)), _full((HID, HID))]
    out_shape = [jax.ShapeDtypeStruct((N, HID), f32),
                 jax.ShapeDtypeStruct((N, 1), f32),
                 jax.ShapeDtypeStruct((N, HID), f32),
                 jax.ShapeDtypeStruct((N, HID), f32)]
    out_specs = [_nblk(HID), _nblk(1), _nblk(HID), _nblk(HID)]

    def body(ar, nr, qr, tr, emb, wn, bn, wt1, bt1, wt2, bt2, qe, wq, bq,
             wi, bi, wa, wb, h_o, qm_o, pa_o, pb_o):
        P = _dot16(_silu(emb[...]), wn[...])
        oh_a = (ar[...] == lax.broadcasted_iota(jnp.int32, (1, 100), 1))
        h_node = _dot16(oh_a.astype(f32), P) + bn[...]
        freq = jnp.exp(lax.broadcasted_iota(jnp.int32, (1, 16), 1).astype(f32)
                       * (-math.log(10000.0) / 15.0))
        pe = tr[...] * freq
        temb0 = jnp.concatenate([jnp.sin(pe), jnp.cos(pe)], axis=1)
        temb = _dot16(_silu(_dot16(temb0, wt1[...]) + bt1[...]),
                      wt2[...]) + bt2[...]
        oh_b = (nr[...] == lax.broadcasted_iota(jnp.int32, (1, 32), 1))
        temb_node = _dot16(oh_b.astype(f32), temb)
        Q = _dot16(_silu(qe[...]), wq[...])
        isq1 = qr[...] == 1
        isqf = isq1.astype(f32)
        qrow = jnp.where(isq1, Q[1:2, :], Q[0:1, :]) + bq[...]
        h = (_dot16(h_node, wi[0:64, :]) + _dot16(temb_node, wi[64:96, :])
             + _dot16(qrow, wi[96:128, :]) + bi[...])
        h_o[...] = h
        qm_o[...] = isqf
        pa_o[...] = _dot16(h, wa[...])
        pb_o[...] = _dot16(h, wb[...])

    return pl.pallas_call(body, grid=grid, in_specs=win, out_specs=out_specs,
                          out_shape=out_shape)(atom_t, nb, isq, tcol,
                                               atom_emb, Wn, bnr, Wt1, bt1r,
                                               Wt2, bt2r, q_emb, Wq, bqr,
                                               Win, binr, Wa0, Wb0)


def kernel(pos, atom_emb, Wn, bn, Wt1, bt1, Wt2, bt2, q_emb, Wq, bq, We1,
           be1, We2, be2, Win, bin_, Wm1, bm1, Wm2, bm2, Watt, batt, Wx, bx,
           Wh1, bh1, Wh2, bh2, atom_type, edge_index, node_batch, is_query,
           t):
    L = Wm1.shape[0]
    src = edge_index[0]
    dst = edge_index[1]
    xpad0 = jnp.pad(pos, ((0, 0), (0, XP - 3)))
    zeros = jnp.zeros((N, ACC), f32)
    row = lambda v: v.reshape(1, -1)
    Wa = [Wm1[l, 0:64] for l in range(L)]
    Wb = [Wm1[l, 64:128] for l in range(L)]
    wd2 = [Wm1[l, 128:129] for l in range(L)]
    Wd = [Wm1[l, 129:193] for l in range(L)]

    h, qm, pha, phb = _prep_call(
        atom_type.reshape(N, 1), node_batch.reshape(N, 1),
        is_query.reshape(N, 1), t.astype(f32).reshape(32, 1),
        atom_emb, Wn, row(bn), Wt1, row(bt1), Wt2, row(bt2), q_emb, Wq,
        row(bq), Win, row(bin_), Wa[0], Wb[0])

    sc_gather, sc_scatter = _sc_kernels()
    xpad = xpad0
    ea16 = None
    for l in range(L):
        hs, hd, xs, xd = sc_gather(pha, phb, xpad, src, dst)
        macc, ea_new = _edge_call(
            l == 0, hs, hd, xs, xd, ea16, row(We1), row(be1), We2, Wd[l],
            wd2[l], row(bm1[l]), row(be2), Wm2[l], row(bm2[l]), Watt[l],
            batt[l].reshape(1, 1), Wx[l], bx[l].reshape(1, 1))
        if l == 0:
            ea16 = ea_new
        accs = sc_scatter(macc, dst, zeros)
        last = l == L - 1
        if last:
            vt_pad = _node_call(True, None, xpad, accs[0], accs[1], qm,
                                xpad0, None, None, None, None, None, None,
                                None)
        else:
            h, xpad, pha, phb = _node_call(
                False, h, xpad, accs[0], accs[1], qm, None, Wh1[l, 0:64],
                Wh1[l, 64:128], row(bh1[l]), Wh2[l], row(bh2[l]), Wa[l + 1],
                Wb[l + 1])
    return vt_pad[:, :3]


# speedup vs baseline: 2.9909x; 1.5554x over previous
import functools
import math

import jax
import jax.numpy as jnp
from jax import lax
from jax.experimental import pallas as pl
from jax.experimental.pallas import tpu as pltpu
from jax.experimental.pallas import tpu_sc as plsc

N = 10000
E = 320000
HID = 64
XP = 16
ACC = HID + XP
AVG_DEG = 32.0

NC, NS = 2, 16
NW = NC * NS
EPW = E // NW
CH = 80
NCHUNK = EPW // CH
DEP = 5
NR = NCHUNK // DEP
NPC = N // NS

BE = 2000
BN = 1000

f32 = jnp.float32


def _silu(x):
    return x * jax.nn.sigmoid(x)


def _bf(x):
    return x.astype(jnp.bfloat16)


def _dot16(a, b):
    return jnp.dot(_bf(a), _bf(b), preferred_element_type=f32)


@functools.cache
def _sc_kernels():
    mesh = plsc.VectorSubcoreMesh(core_axis_name="c", subcore_axis_name="s")
    cp = pltpu.CompilerParams(use_tc_tiling_on_sc=False)

    gather_scratch = (
        [pltpu.VMEM((EPW,), jnp.int32)] * 2
        + [pltpu.VMEM((CH, ACC), f32)] * (2 * DEP)
        + [pltpu.SemaphoreType.DMA] * (4 * DEP)
    )

    @functools.partial(
        pl.kernel,
        compiler_params=cp,
        out_type=(
            jax.ShapeDtypeStruct((E, ACC), f32),
            jax.ShapeDtypeStruct((E, ACC), f32),
        ),
        mesh=mesh,
        scratch_types=gather_scratch,
    )
    def sc_gather(ta, tb, src, dst, gso, gdo, *scr):
        ix_s, ix_d = scr[0], scr[1]
        rs = scr[2:2 + DEP]
        rd = scr[2 + DEP:2 + 2 * DEP]
        sg = scr[2 + 2 * DEP:2 + 3 * DEP]
        dg = scr[2 + 3 * DEP:2 + 4 * DEP]
        sw = scr[2 + 4 * DEP:2 + 5 * DEP]
        dw = scr[2 + 5 * DEP:2 + 6 * DEP]
        wid = lax.axis_index("s") * NC + lax.axis_index("c")
        base = wid * EPW
        pltpu.sync_copy(src.at[pl.ds(base, EPW)], ix_s)
        pltpu.sync_copy(dst.at[pl.ds(base, EPW)], ix_d)

        def fire(i, b):
            pltpu.async_copy(ta.at[ix_s.at[pl.ds(i * CH, CH)]], rs[b], sg[b])
            pltpu.async_copy(tb.at[ix_d.at[pl.ds(i * CH, CH)]], rd[b], dg[b])

        for b in range(DEP):
            fire(b, b)

        @pl.loop(0, NR)
        def _(j):
            for b in range(DEP):
                i = j * DEP + b
                off = base + i * CH
                pltpu.make_async_copy(
                    ta.at[ix_s.at[pl.ds(i * CH, CH)]], rs[b], sg[b]).wait()
                pltpu.async_copy(rs[b], gso.at[pl.ds(off, CH)], sw[b])
                pltpu.make_async_copy(
                    tb.at[ix_d.at[pl.ds(i * CH, CH)]], rd[b], dg[b]).wait()
                pltpu.async_copy(rd[b], gdo.at[pl.ds(off, CH)], dw[b])

            @pl.when(j < NR - 1)
            def _():
                for b in range(DEP):
                    i = j * DEP + b
                    off = base + i * CH
                    pltpu.make_async_copy(
                        rs[b], gso.at[pl.ds(off, CH)], sw[b]).wait()
                    pltpu.make_async_copy(
                        rd[b], gdo.at[pl.ds(off, CH)], dw[b]).wait()
                    fire((j + 1) * DEP + b, b)

        for b in range(DEP):
            off = base + ((NR - 1) * DEP + b) * CH
            pltpu.make_async_copy(rs[b], gso.at[pl.ds(off, CH)], sw[b]).wait()
            pltpu.make_async_copy(rd[b], gdo.at[pl.ds(off, CH)], dw[b]).wait()

    scatter_scratch = (
        [pltpu.VMEM((CH,), jnp.int32)] * DEP
        + [pltpu.VMEM((CH, ACC), f32)] * DEP
        + [pltpu.SemaphoreType.DMA] * (2 * DEP)
        + [pltpu.VMEM_SHARED((N, ACC), f32)]
    )

    @functools.partial(
        pl.kernel,
        compiler_params=cp,
        out_type=jax.ShapeDtypeStruct((NC, N, ACC), f32),
        mesh=mesh,
        scratch_types=scatter_scratch,
    )
    def sc_scatter(macc, dstidx, zeros, acc_out, *scr):
        ib = scr[0:DEP]
        pb = scr[DEP:2 * DEP]
        ps = scr[2 * DEP:3 * DEP]
        ss = scr[3 * DEP:4 * DEP]
        acc_sh = scr[4 * DEP]
        cid = lax.axis_index("c")
        sid = lax.axis_index("s")
        wid = sid * NC + cid
        pltpu.sync_copy(zeros.at[pl.ds(sid * NPC, NPC)],
                        acc_sh.at[pl.ds(sid * NPC, NPC)])
        plsc.subcore_barrier()
        base = wid * EPW

        def fire(i, b):
            off = base + i * CH
            pltpu.sync_copy(dstidx.at[pl.ds(off, CH)], ib[b])
            pltpu.async_copy(macc.at[pl.ds(off, CH)], pb[b], ps[b])

        for b in range(DEP):
            fire(b, b)

        @pl.loop(0, NR)
        def _(j):
            for b in range(DEP):
                i = j * DEP + b
                off = base + i * CH
                pltpu.make_async_copy(
                    macc.at[pl.ds(off, CH)], pb[b], ps[b]).wait()
                pltpu.async_copy(pb[b], acc_sh.at[ib[b]], ss[b], add=True)

            @pl.when(j < NR - 1)
            def _():
                for b in range(DEP):
                    pltpu.make_async_copy(
                        pb[b], acc_sh.at[ib[b]], ss[b]).wait()
                    fire((j + 1) * DEP + b, b)

        for b in range(DEP):
            pltpu.make_async_copy(pb[b], acc_sh.at[ib[b]], ss[b]).wait()
        plsc.subcore_barrier()
        pltpu.sync_copy(acc_sh.at[pl.ds(sid * NPC, NPC)],
                        acc_out.at[cid].at[pl.ds(sid * NPC, NPC)])

    return sc_gather, sc_scatter


def _full(shape):
    return pl.BlockSpec(shape, lambda i: tuple(0 for _ in shape))


def _blk(d):
    return pl.BlockSpec((BE, d), lambda i: (i, 0))


def _nblk(d):
    return pl.BlockSpec((BN, d), lambda i: (i, 0))


def _edge_body(first, gs, gd, ear, we1, be1, We2, Wd, wd2, bm1r,
               be2, Wm2, bm2r, Watt, battr, Wx, bxr, macc, eaout):
    gsv = gs[...]
    gdv = gd[...]
    hsum = gsv[:, :HID] + gdv[:, :HID]
    diff = gsv[:, HID:] - gdv[:, HID:]
    d2 = jnp.sum(diff * diff, axis=1, keepdims=True)
    if first:
        d0 = jnp.sqrt(d2 + 1e-8)
        u = _silu(d0 * we1[...] + be1[...])
        ea = _bf(_dot16(u, We2[...]) + be2[...])
        eaout[...] = ea
    else:
        ea = ear[...]
    d2c = _bf(d2).astype(f32) * _bf(wd2[...]).astype(f32)
    pre1 = (hsum + d2c
            + jnp.dot(ea, _bf(Wd[...]), preferred_element_type=f32)
            + bm1r[...])
    m = _silu(_dot16(_silu(pre1), Wm2[...]) + bm2r[...])
    att = jax.nn.sigmoid(_dot16(m, Watt[...]) + battr[...])
    mt = m * att
    coef = jnp.tanh(_dot16(mt, Wx[...]) + bxr[...])
    macc[...] = jnp.concatenate([mt, diff * coef], axis=1)


def _edge_call(first, gs, gd, ea16, we1r, be1r, We2, Wd, wd2r, bm1r,
               be2r, Wm2, bm2r, Watt, battr, Wx, bxr):
    grid = (E // BE,)
    win = [_blk(ACC), _blk(ACC)]
    args = [gs, gd]
    if not first:
        win.append(_blk(HID))
        args.append(ea16)
    win += [_full((1, HID)), _full((1, HID)), _full((HID, HID)),
            _full((HID, HID)), _full((1, HID)), _full((1, HID)),
            _full((1, HID)), _full((HID, HID)), _full((1, HID)),
            _full((HID, 1)), _full((1, 1)), _full((HID, 1)), _full((1, 1))]
    args += [we1r, be1r, We2, Wd, wd2r, bm1r, be2r, Wm2, bm2r, Watt, battr,
             Wx, bxr]
    out_shape = [jax.ShapeDtypeStruct((E, ACC), f32)]
    out_specs = [_blk(ACC)]
    if first:
        out_shape.append(jax.ShapeDtypeStruct((E, HID), jnp.bfloat16))
        out_specs.append(_blk(HID))

    def body(*refs):
        ins = refs[:2]
        k = 2
        ear = None
        if not first:
            ear = refs[2]
            k = 3
        w = refs[k:k + 13]
        outs = refs[k + 13:]
        _edge_body(first, *ins, ear, *w, outs[0],
                   outs[1] if first else None)

    res = pl.pallas_call(
        body, grid=grid, in_specs=win, out_specs=out_specs,
        out_shape=out_shape)(*args)
    return res if first else (res[0], None)


def _node_call(last, h, xpad, acc0, acc1, qm, pospad, Wh1h, Wh1a, bh1r, Wh2,
               bh2r, Wa, Wb):
    grid = (N // BN,)
    if last:
        args = [xpad, acc0, acc1, qm, pospad]
        win = [_nblk(XP), _nblk(ACC), _nblk(ACC), _nblk(1), _nblk(XP)]
        out_shape = jax.ShapeDtypeStruct((N, XP), f32)
        out_specs = _nblk(XP)

        def body(xr, a0, a1, qr, pr, vt):
            dx = ((a0[..., HID:] + a1[..., HID:]) * (1.0 / AVG_DEG)) * qr[...]
            vt[...] = xr[...] + dx - pr[...]
    else:
        args = [h, xpad, acc0, acc1, qm, Wh1h, Wh1a, bh1r, Wh2, bh2r, Wa, Wb]
        win = [_nblk(HID), _nblk(XP), _nblk(ACC), _nblk(ACC), _nblk(1),
               _full((HID, HID)), _full((HID, HID)), _full((1, HID)),
               _full((HID, HID)), _full((1, HID)), _full((HID, HID)),
               _full((HID, HID))]
        out_shape = [jax.ShapeDtypeStruct((N, HID), f32),
                     jax.ShapeDtypeStruct((N, XP), f32),
                     jax.ShapeDtypeStruct((N, ACC), f32),
                     jax.ShapeDtypeStruct((N, ACC), f32)]
        out_specs = [_nblk(HID), _nblk(XP), _nblk(ACC), _nblk(ACC)]

        def body(hr, xr, a0, a1, qr, w1h, w1a, b1, w2, b2, wa, wb,
                 ho, xo, tao, tbo):
            dx = ((a0[..., HID:] + a1[..., HID:]) * (1.0 / AVG_DEG)) * qr[...]
            x_new = xr[...] + dx
            agg = a0[..., :HID] + a1[..., :HID]
            hh = _silu(_dot16(hr[...], w1h[...]) + _dot16(agg, w1a[...])
                       + b1[...])
            h_new = hr[...] + _dot16(hh, w2[...]) + b2[...]
            ho[...] = h_new
            xo[...] = x_new
            tao[...] = jnp.concatenate([_dot16(h_new, wa[...]), x_new], axis=1)
            tbo[...] = jnp.concatenate([_dot16(h_new, wb[...]), x_new], axis=1)

    return pl.pallas_call(body, grid=grid, in_specs=win, out_specs=out_specs,
                          out_shape=out_shape)(*args)


def _prep_call(atom_t, nb, isq, tcol, xpad0, atom_emb, Wn, bnr, Wt1, bt1r,
               Wt2, bt2r, q_emb, Wq, bqr, Win, binr, Wa0, Wb0):
    grid = (N // BN,)
    win = [_nblk(1), _nblk(1), _nblk(1), _full((32, 1)), _nblk(XP),
           _full((100, HID)), _full((HID, HID)), _full((1, HID)),
           _full((32, 32)), _full((1, 32)), _full((32, 32)), _full((1, 32)),
           _full((2, 32)), _full((32, 32)), _full((1, 32)),
           _full((128, HID)), _full((1, HID)),
           _full((HID, HID)), _full((HID, HID))]
    out_shape = [jax.ShapeDtypeStruct((N, HID), f32),
                 jax.ShapeDtypeStruct((N, 1), f32),
                 jax.ShapeDtypeStruct((N, ACC), f32),
                 jax.ShapeDtypeStruct((N, ACC), f32)]
    out_specs = [_nblk(HID), _nblk(1), _nblk(ACC), _nblk(ACC)]

    def body(ar, nr, qr, tr, xr, emb, wn, bn, wt1, bt1, wt2, bt2, qe, wq, bq,
             wi, bi, wa, wb, h_o, qm_o, ta_o, tb_o):
        P = _dot16(_silu(emb[...]), wn[...])
        oh_a = (ar[...] == lax.broadcasted_iota(jnp.int32, (1, 100), 1))
        h_node = _dot16(oh_a.astype(f32), P) + bn[...]
        freq = jnp.exp(lax.broadcasted_iota(jnp.int32, (1, 16), 1).astype(f32)
                       * (-math.log(10000.0) / 15.0))
        pe = tr[...] * freq
        temb0 = jnp.concatenate([jnp.sin(pe), jnp.cos(pe)], axis=1)
        temb = _dot16(_silu(_dot16(temb0, wt1[...]) + bt1[...]),
                      wt2[...]) + bt2[...]
        oh_b = (nr[...] == lax.broadcasted_iota(jnp.int32, (1, 32), 1))
        temb_node = _dot16(oh_b.astype(f32), temb)
        Q = _dot16(_silu(qe[...]), wq[...])
        isq1 = qr[...] == 1
        isqf = isq1.astype(f32)
        qrow = jnp.where(isq1, Q[1:2, :], Q[0:1, :]) + bq[...]
        h = (_dot16(h_node, wi[0:64, :]) + _dot16(temb_node, wi[64:96, :])
             + _dot16(qrow, wi[96:128, :]) + bi[...])
        h_o[...] = h
        qm_o[...] = isqf
        xv = xr[...]
        ta_o[...] = jnp.concatenate([_dot16(h, wa[...]), xv], axis=1)
        tb_o[...] = jnp.concatenate([_dot16(h, wb[...]), xv], axis=1)

    return pl.pallas_call(body, grid=grid, in_specs=win, out_specs=out_specs,
                          out_shape=out_shape)(atom_t, nb, isq, tcol, xpad0,
                                               atom_emb, Wn, bnr, Wt1, bt1r,
                                               Wt2, bt2r, q_emb, Wq, bqr,
                                               Win, binr, Wa0, Wb0)


def kernel(pos, atom_emb, Wn, bn, Wt1, bt1, Wt2, bt2, q_emb, Wq, bq, We1,
           be1, We2, be2, Win, bin_, Wm1, bm1, Wm2, bm2, Watt, batt, Wx, bx,
           Wh1, bh1, Wh2, bh2, atom_type, edge_index, node_batch, is_query,
           t):
    L = Wm1.shape[0]
    src = edge_index[0]
    dst = edge_index[1]
    xpad0 = jnp.pad(pos, ((0, 0), (0, XP - 3)))
    zeros = jnp.zeros((N, ACC), f32)
    row = lambda v: v.reshape(1, -1)
    Wa = [Wm1[l, 0:64] for l in range(L)]
    Wb = [Wm1[l, 64:128] for l in range(L)]
    wd2 = [Wm1[l, 128:129] for l in range(L)]
    Wd = [Wm1[l, 129:193] for l in range(L)]

    h, qm, ta, tb = _prep_call(
        atom_type.reshape(N, 1), node_batch.reshape(N, 1),
        is_query.reshape(N, 1), t.astype(f32).reshape(32, 1), xpad0,
        atom_emb, Wn, row(bn), Wt1, row(bt1), Wt2, row(bt2), q_emb, Wq,
        row(bq), Win, row(bin_), Wa[0], Wb[0])

    sc_gather, sc_scatter = _sc_kernels()
    xpad = xpad0
    ea16 = None
    for l in range(L):
        gs, gd = sc_gather(ta, tb, src, dst)
        macc, ea_new = _edge_call(
            l == 0, gs, gd, ea16, row(We1), row(be1), We2, Wd[l],
            wd2[l], row(bm1[l]), row(be2), Wm2[l], row(bm2[l]), Watt[l],
            batt[l].reshape(1, 1), Wx[l], bx[l].reshape(1, 1))
        if l == 0:
            ea16 = ea_new
        accs = sc_scatter(macc, dst, zeros)
        last = l == L - 1
        if last:
            vt_pad = _node_call(True, None, xpad, accs[0], accs[1], qm,
                                xpad0, None, None, None, None, None, None,
                                None)
        else:
            h, xpad, ta, tb = _node_call(
                False, h, xpad, accs[0], accs[1], qm, None, Wh1[l, 0:64],
                Wh1[l, 64:128], row(bh1[l]), Wh2[l], row(bh2[l]), Wa[l + 1],
                Wb[l + 1])
    return vt_pad[:, :3]
